# baseline (speedup 1.0000x reference)
import jax
import jax.numpy as jnp
from jax.experimental import pallas as pl
from jax.experimental.pallas import tpu as pltpu

_DIST2_THRESH = 0.2 * 0.2
_CONF_THRESH = 0.7
_BIG = 1e30
_BS = 4
_STRIP = 64


def _adj_kernel(in_ref, out_ref, odiag_ref, tcols_ref):
    n = out_ref.shape[2]

    @pl.when(pl.program_id(0) == 0)
    def _init():
        rows = jax.lax.broadcasted_iota(jnp.int32, (n, n), 0)
        cols = jax.lax.broadcasted_iota(jnp.int32, (n, n), 1)
        odiag_ref[...] = (rows != cols).astype(jnp.float32)

    for s in range(_BS):
        x = in_ref[s, 0:1, :]
        y = in_ref[s, 1:2, :]
        z = in_ref[s, 2:3, :]
        t_row = in_ref[s, 3:4, :]
        tcols_ref[:, 0:1] = jnp.transpose(in_ref[s, 0:1, :])
        tcols_ref[:, 1:2] = jnp.transpose(in_ref[s, 1:2, :])
        tcols_ref[:, 2:3] = jnp.transpose(in_ref[s, 2:3, :])
        tcols_ref[:, 3:4] = jnp.transpose(in_ref[s, 4:5, :])
        for st in range(n // _STRIP):
            s0 = st * _STRIP
            xc = tcols_ref[s0 : s0 + _STRIP, 0:1]
            yc = tcols_ref[s0 : s0 + _STRIP, 1:2]
            zc = tcols_ref[s0 : s0 + _STRIP, 2:3]
            tc = tcols_ref[s0 : s0 + _STRIP, 3:4]
            g = xc * x + yc * y + zc * z
            t = tc + t_row
            out_ref[s, s0 : s0 + _STRIP, :] = jnp.where(
                g < t, odiag_ref[s0 : s0 + _STRIP, :], 0.0
            )


def kernel(centroid, obj_conf):
    B, N, _ = centroid.shape
    n2 = jnp.sum(centroid * centroid, axis=-1)
    conf_ok = obj_conf >= _CONF_THRESH
    half = (n2 - 0.5 * _DIST2_THRESH) * 0.5
    t_row = jnp.where(conf_ok, half, -jnp.inf)
    dist_enabled = (jnp.arange(B)[:, None] >= 2) & (jnp.arange(N)[None, :] >= 2)
    t_col = jnp.where(conf_ok, jnp.where(dist_enabled, half, _BIG), -jnp.inf)
    packed = jnp.concatenate(
        [
            jnp.transpose(centroid, (0, 2, 1)),
            t_row[:, None, :],
            t_col[:, None, :],
        ],
        axis=1,
    )
    return pl.pallas_call(
        _adj_kernel,
        grid=(B // _BS,),
        in_specs=[pl.BlockSpec((_BS, 5, N), lambda b: (b, 0, 0))],
        out_specs=pl.BlockSpec((_BS, N, N), lambda b: (b, 0, 0)),
        out_shape=jax.ShapeDtypeStruct((B, N, N), jnp.float32),
        scratch_shapes=[
            pltpu.VMEM((N, N), jnp.float32),
            pltpu.VMEM((N, 8), jnp.float32),
        ],
    )(packed)


# speedup vs baseline: 1.4800x; 1.4800x over previous
import jax
import jax.numpy as jnp
from jax.experimental import pallas as pl
from jax.experimental.pallas import tpu as pltpu

_DIST2_THRESH = 0.2 * 0.2
_CONF_THRESH = 0.7
_BIG = 1e30
_BS = 8


def _adj_kernel(in_ref, out_ref, odiag_ref):
    n = out_ref.shape[2]

    @pl.when(pl.program_id(0) == 0)
    def _init():
        rows = jax.lax.broadcasted_iota(jnp.int32, (n, n), 0)
        cols = jax.lax.broadcasted_iota(jnp.int32, (n, n), 1)
        odiag_ref[...] = (rows != cols).astype(jnp.float32)

    od = odiag_ref[...]
    for s in range(_BS):
        x = in_ref[s, 0:1, :]
        y = in_ref[s, 1:2, :]
        z = in_ref[s, 2:3, :]
        t_row = in_ref[s, 3:4, :]
        xc = jnp.transpose(in_ref[s, 0:1, :])
        yc = jnp.transpose(in_ref[s, 1:2, :])
        zc = jnp.transpose(in_ref[s, 2:3, :])
        tc = jnp.transpose(in_ref[s, 4:5, :])
        g = xc * x + yc * y + zc * z
        t = tc + t_row
        out_ref[s] = jnp.where(g < t, od, 0.0)


def kernel(centroid, obj_conf):
    B, N, _ = centroid.shape
    n2 = jnp.sum(centroid * centroid, axis=-1)
    conf_ok = obj_conf >= _CONF_THRESH
    half = (n2 - 0.5 * _DIST2_THRESH) * 0.5
    t_row = jnp.where(conf_ok, half, -jnp.inf)
    dist_enabled = (jnp.arange(B)[:, None] >= 2) & (jnp.arange(N)[None, :] >= 2)
    t_col = jnp.where(conf_ok, jnp.where(dist_enabled, half, _BIG), -jnp.inf)
    packed = jnp.concatenate(
        [
            jnp.transpose(centroid, (0, 2, 1)),
            t_row[:, None, :],
            t_col[:, None, :],
        ],
        axis=1,
    )
    return pl.pallas_call(
        _adj_kernel,
        grid=(B // _BS,),
        in_specs=[pl.BlockSpec((_BS, 5, N), lambda b: (b, 0, 0))],
        out_specs=pl.BlockSpec((_BS, N, N), lambda b: (b, 0, 0)),
        out_shape=jax.ShapeDtypeStruct((B, N, N), jnp.float32),
        scratch_shapes=[pltpu.VMEM((N, N), jnp.float32)],
    )(packed)


# speedup vs baseline: 1.5949x; 1.0776x over previous
import jax
import jax.numpy as jnp
from jax.experimental import pallas as pl
from jax.experimental.pallas import tpu as pltpu

_DIST2_THRESH = 0.2 * 0.2
_CONF_THRESH = 0.7
_BIG = 1e30
_BS = 4
_CHUNK = 128


def _adj_kernel(in_ref, out_ref, odiag_ref):
    n = out_ref.shape[2]

    @pl.when(pl.program_id(0) == 0)
    def _init():
        rows = jax.lax.broadcasted_iota(jnp.int32, (n, n), 0)
        cols = jax.lax.broadcasted_iota(jnp.int32, (n, n), 1)
        odiag_ref[...] = (rows != cols).astype(jnp.float32)

    for s in range(_BS):
        x = in_ref[s, 0:1, :]
        y = in_ref[s, 1:2, :]
        z = in_ref[s, 2:3, :]
        t_row = in_ref[s, 3:4, :]
        for h in range(n // _CHUNK):
            c0 = h * _CHUNK
            xc = jnp.transpose(in_ref[s, 0:1, c0 : c0 + _CHUNK])
            yc = jnp.transpose(in_ref[s, 1:2, c0 : c0 + _CHUNK])
            zc = jnp.transpose(in_ref[s, 2:3, c0 : c0 + _CHUNK])
            tc = jnp.transpose(in_ref[s, 4:5, c0 : c0 + _CHUNK])
            g = xc * x + yc * y + zc * z
            t = tc + t_row
            out_ref[s, c0 : c0 + _CHUNK, :] = jnp.where(
                g < t, odiag_ref[c0 : c0 + _CHUNK, :], 0.0
            )


def kernel(centroid, obj_conf):
    B, N, _ = centroid.shape
    n2 = jnp.sum(centroid * centroid, axis=-1)
    conf_ok = obj_conf >= _CONF_THRESH
    half = (n2 - 0.5 * _DIST2_THRESH) * 0.5
    t_row = jnp.where(conf_ok, half, -jnp.inf)
    dist_enabled = (jnp.arange(B)[:, None] >= 2) & (jnp.arange(N)[None, :] >= 2)
    t_col = jnp.where(conf_ok, jnp.where(dist_enabled, half, _BIG), -jnp.inf)
    packed = jnp.concatenate(
        [
            jnp.transpose(centroid, (0, 2, 1)),
            t_row[:, None, :],
            t_col[:, None, :],
        ],
        axis=1,
    )
    return pl.pallas_call(
        _adj_kernel,
        grid=(B // _BS,),
        in_specs=[pl.BlockSpec((_BS, 5, N), lambda b: (b, 0, 0))],
        out_specs=pl.BlockSpec((_BS, N, N), lambda b: (b, 0, 0)),
        out_shape=jax.ShapeDtypeStruct((B, N, N), jnp.float32),
        scratch_shapes=[pltpu.VMEM((N, N), jnp.float32)],
    )(packed)
